# baseline (speedup 1.0000x reference)
import functools

import jax
import jax.numpy as jnp
from jax import lax
from jax.experimental import pallas as pl
from jax.experimental.pallas import tpu as pltpu
from jax.experimental.pallas import tpu_sc as plsc

_NUM_CORES = 2
_NUM_SUBCORES = 16
_NW = _NUM_CORES * _NUM_SUBCORES

_BATCH = 16384
_DIM = 32
_ZONES = 9
_BPW = _BATCH // _NW
_CHUNKS = _BPW // 16


@functools.partial(
    pl.kernel,
    out_type=jax.ShapeDtypeStruct((_DIM, _BATCH), jnp.float32),
    mesh=plsc.VectorSubcoreMesh(
        core_axis_name="c",
        subcore_axis_name="s",
        num_cores=_NUM_CORES,
        num_subcores=_NUM_SUBCORES,
    ),
    scratch_types=[
        pltpu.VMEM((_BPW,), jnp.int32),
        pltpu.VMEM((_ZONES, _DIM), jnp.float32),
        pltpu.VMEM((_ZONES * _DIM,), jnp.float32),
        pltpu.VMEM((_DIM, _BPW), jnp.float32),
    ],
    compiler_params=pltpu.CompilerParams(
        needs_layout_passes=False, skip_device_barrier=True
    ),
)
def _gather_kernel(idx_hbm, table_hbm, out_hbm, idx_v, table_v, table_f, rows_v):
    wid = lax.axis_index("s") * _NUM_CORES + lax.axis_index("c")
    base = wid * _BPW
    pltpu.sync_copy(idx_hbm.at[pl.ds(base, _BPW)], idx_v)
    pltpu.sync_copy(table_hbm, table_v)
    for r in range(_ZONES):
        for h in range(_DIM // 16):
            table_f[pl.ds(r * _DIM + h * 16, 16)] = table_v[r, pl.ds(h * 16, 16)]

    @plsc.parallel_loop(0, _CHUNKS, 1, unroll=2)
    def body(i):
        rowbase = idx_v[pl.ds(i * 16, 16)] * _DIM
        for d in range(_DIM):
            vals = plsc.load_gather(table_f, [rowbase + d])
            rows_v[d, pl.ds(i * 16, 16)] = vals

    pltpu.sync_copy(rows_v, out_hbm.at[:, pl.ds(base, _BPW)])


def kernel(zone_idx, embedding_table):
    out_t = _gather_kernel(zone_idx.astype(jnp.int32), embedding_table)
    return out_t.T


# speedup vs baseline: 1.0053x; 1.0053x over previous
import functools

import jax
import jax.numpy as jnp
from jax import lax
from jax.experimental import pallas as pl
from jax.experimental.pallas import tpu as pltpu
from jax.experimental.pallas import tpu_sc as plsc

_NUM_CORES = 2
_NUM_SUBCORES = 16
_NW = _NUM_CORES * _NUM_SUBCORES

_BATCH = 16384
_DIM = 32
_ZONES = 9
_BPW = _BATCH // _NW
_CHUNKS = _BPW // 16


@functools.partial(
    pl.kernel,
    out_type=jax.ShapeDtypeStruct((_DIM, _BATCH), jnp.float32),
    mesh=plsc.VectorSubcoreMesh(
        core_axis_name="c",
        subcore_axis_name="s",
        num_cores=_NUM_CORES,
        num_subcores=_NUM_SUBCORES,
    ),
    scratch_types=[
        pltpu.VMEM((_BPW,), jnp.int32),
        pltpu.VMEM((_ZONES, _DIM), jnp.float32),
        pltpu.VMEM((_ZONES * _DIM,), jnp.float32),
        pltpu.VMEM((_DIM, _BPW), jnp.float32),
    ],
    compiler_params=pltpu.CompilerParams(
        needs_layout_passes=False
    ),
)
def _gather_kernel(idx_hbm, table_hbm, out_hbm, idx_v, table_v, table_f, rows_v):
    wid = lax.axis_index("s") * _NUM_CORES + lax.axis_index("c")
    base = wid * _BPW
    pltpu.sync_copy(idx_hbm.at[pl.ds(base, _BPW)], idx_v)
    pltpu.sync_copy(table_hbm, table_v)
    for r in range(_ZONES):
        for h in range(_DIM // 16):
            table_f[pl.ds(r * _DIM + h * 16, 16)] = table_v[r, pl.ds(h * 16, 16)]

    @plsc.parallel_loop(0, _CHUNKS, 1, unroll=1)
    def body(i):
        rowbase = idx_v[pl.ds(i * 16, 16)] * _DIM
        for d in range(_DIM):
            vals = plsc.load_gather(table_f, [rowbase + d])
            rows_v[d, pl.ds(i * 16, 16)] = vals

    pltpu.sync_copy(rows_v, out_hbm.at[:, pl.ds(base, _BPW)])


def kernel(zone_idx, embedding_table):
    out_t = _gather_kernel(zone_idx.astype(jnp.int32), embedding_table)
    return out_t.T


# speedup vs baseline: 1.2525x; 1.2459x over previous
import functools

import jax
import jax.numpy as jnp
from jax import lax
from jax.experimental import pallas as pl
from jax.experimental.pallas import tpu as pltpu
from jax.experimental.pallas import tpu_sc as plsc

_NUM_CORES = 2
_NUM_SUBCORES = 16
_NW = _NUM_CORES * _NUM_SUBCORES

_BATCH = 16384
_DIM = 32
_ZONES = 9
_BPW = _BATCH // _NW
_CHUNKS = _BPW // 16


@functools.partial(
    pl.kernel,
    out_type=jax.ShapeDtypeStruct((_DIM, _BATCH), jnp.float32),
    mesh=plsc.VectorSubcoreMesh(
        core_axis_name="c",
        subcore_axis_name="s",
        num_cores=_NUM_CORES,
        num_subcores=_NUM_SUBCORES,
    ),
    scratch_types=[
        pltpu.VMEM((_BPW,), jnp.int32),
        pltpu.VMEM((_ZONES, _DIM), jnp.float32),
        pltpu.VMEM((_DIM, 16), jnp.float32),
        pltpu.VMEM((_DIM, _BPW), jnp.float32),
    ],
    compiler_params=pltpu.CompilerParams(
        needs_layout_passes=False
    ),
)
def _gather_kernel(idx_hbm, table_hbm, out_hbm, idx_v, table_v, ttv, rows_v):
    wid = lax.axis_index("s") * _NUM_CORES + lax.axis_index("c")
    base = wid * _BPW
    pltpu.sync_copy(idx_hbm.at[pl.ds(base, _BPW)], idx_v)
    pltpu.sync_copy(table_hbm, table_v)
    lane = lax.iota(jnp.int32, 16)
    for r in range(_ZONES):
        for h in range(_DIM // 16):
            vals = table_v[r, pl.ds(h * 16, 16)]
            plsc.store_scatter(
                ttv, [lane + h * 16, jnp.full((16,), r, dtype=jnp.int32)], vals
            )

    @plsc.parallel_loop(0, _CHUNKS, 1, unroll=1)
    def body(i):
        rowidx = idx_v[pl.ds(i * 16, 16)]
        for d in range(_DIM):
            tcol = ttv[d]
            rows_v[d, pl.ds(i * 16, 16)] = tcol.at[rowidx].get(
                mode="promise_in_bounds"
            )

    pltpu.sync_copy(rows_v, out_hbm.at[:, pl.ds(base, _BPW)])


def kernel(zone_idx, embedding_table):
    out_t = _gather_kernel(zone_idx.astype(jnp.int32), embedding_table)
    return out_t.T


# speedup vs baseline: 1.2583x; 1.0046x over previous
import functools

import jax
import jax.numpy as jnp
from jax import lax
from jax.experimental import pallas as pl
from jax.experimental.pallas import tpu as pltpu
from jax.experimental.pallas import tpu_sc as plsc

_NUM_CORES = 2
_NUM_SUBCORES = 16
_NW = _NUM_CORES * _NUM_SUBCORES

_BATCH = 16384
_DIM = 32
_ZONES = 9
_BPW = _BATCH // _NW
_CHUNKS = _BPW // 16


@functools.partial(
    pl.kernel,
    out_type=jax.ShapeDtypeStruct((_DIM, _BATCH), jnp.float32),
    mesh=plsc.VectorSubcoreMesh(
        core_axis_name="c",
        subcore_axis_name="s",
        num_cores=_NUM_CORES,
        num_subcores=_NUM_SUBCORES,
    ),
    scratch_types=[
        pltpu.VMEM((_BPW,), jnp.int32),
        pltpu.VMEM((_ZONES, _DIM), jnp.float32),
        pltpu.VMEM((_DIM, 16), jnp.float32),
        pltpu.VMEM((_DIM, _BPW), jnp.float32),
    ],
    compiler_params=pltpu.CompilerParams(
        needs_layout_passes=False
    ),
)
def _gather_kernel(idx_hbm, table_hbm, out_hbm, idx_v, table_v, ttv, rows_v):
    wid = lax.axis_index("s") * _NUM_CORES + lax.axis_index("c")
    base = wid * _BPW
    pltpu.sync_copy(idx_hbm.at[pl.ds(base, _BPW)], idx_v)
    pltpu.sync_copy(table_hbm, table_v)
    lane = lax.iota(jnp.int32, 16)
    for r in range(_ZONES):
        for h in range(_DIM // 16):
            vals = table_v[r, pl.ds(h * 16, 16)]
            plsc.store_scatter(
                ttv, [lane + h * 16, jnp.full((16,), r, dtype=jnp.int32)], vals
            )

    @plsc.parallel_loop(0, _CHUNKS, 1, unroll=2)
    def body(i):
        rowidx = idx_v[pl.ds(i * 16, 16)]
        for d in range(_DIM):
            tcol = ttv[d]
            rows_v[d, pl.ds(i * 16, 16)] = tcol.at[rowidx].get(
                mode="promise_in_bounds"
            )

    pltpu.sync_copy(rows_v, out_hbm.at[:, pl.ds(base, _BPW)])


def kernel(zone_idx, embedding_table):
    out_t = _gather_kernel(zone_idx.astype(jnp.int32), embedding_table)
    return out_t.T


# speedup vs baseline: 1.2666x; 1.0066x over previous
import functools

import jax
import jax.numpy as jnp
from jax import lax
from jax.experimental import pallas as pl
from jax.experimental.pallas import tpu as pltpu
from jax.experimental.pallas import tpu_sc as plsc

_NUM_CORES = 2
_NUM_SUBCORES = 16
_NW = _NUM_CORES * _NUM_SUBCORES

_BATCH = 16384
_DIM = 32
_ZONES = 9
_BPW = _BATCH // _NW
_CHUNKS = _BPW // 16


@functools.partial(
    pl.kernel,
    out_type=jax.ShapeDtypeStruct((_DIM, _BATCH), jnp.float32),
    mesh=plsc.VectorSubcoreMesh(
        core_axis_name="c",
        subcore_axis_name="s",
        num_cores=_NUM_CORES,
        num_subcores=_NUM_SUBCORES,
    ),
    scratch_types=[
        pltpu.VMEM((_BPW,), jnp.int32),
        pltpu.VMEM((_ZONES, _DIM), jnp.float32),
        pltpu.VMEM((_DIM, 16), jnp.float32),
        pltpu.VMEM((_DIM, _BPW), jnp.float32),
    ],
    compiler_params=pltpu.CompilerParams(
        needs_layout_passes=False
    ),
)
def _gather_kernel(idx_hbm, table_hbm, out_hbm, idx_v, table_v, ttv, rows_v):
    wid = lax.axis_index("s") * _NUM_CORES + lax.axis_index("c")
    base = wid * _BPW
    pltpu.sync_copy(idx_hbm.at[pl.ds(base, _BPW)], idx_v)
    pltpu.sync_copy(table_hbm, table_v)
    lane = lax.iota(jnp.int32, 16)
    for r in range(_ZONES):
        for h in range(_DIM // 16):
            vals = table_v[r, pl.ds(h * 16, 16)]
            plsc.store_scatter(
                ttv, [lane + h * 16, jnp.full((16,), r, dtype=jnp.int32)], vals
            )

    @plsc.parallel_loop(0, _CHUNKS, 1, unroll=1)
    def body(i):
        rowidx = idx_v[pl.ds(i * 16, 16)]

        @plsc.parallel_loop(0, _DIM, 1, unroll=4)
        def cols(d):
            tcol = ttv[d]
            rows_v[d, pl.ds(i * 16, 16)] = tcol.at[rowidx].get(
                mode="promise_in_bounds"
            )

    pltpu.sync_copy(rows_v, out_hbm.at[:, pl.ds(base, _BPW)])


def kernel(zone_idx, embedding_table):
    out_t = _gather_kernel(zone_idx.astype(jnp.int32), embedding_table)
    return out_t.T


# speedup vs baseline: 1.2688x; 1.0017x over previous
import functools

import jax
import jax.numpy as jnp
from jax import lax
from jax.experimental import pallas as pl
from jax.experimental.pallas import tpu as pltpu
from jax.experimental.pallas import tpu_sc as plsc

_NUM_CORES = 2
_NUM_SUBCORES = 16
_NW = _NUM_CORES * _NUM_SUBCORES

_BATCH = 16384
_DIM = 32
_ZONES = 9
_BPW = _BATCH // _NW
_CHUNKS = _BPW // 16


@functools.partial(
    pl.kernel,
    out_type=jax.ShapeDtypeStruct((_DIM, _BATCH), jnp.float32),
    mesh=plsc.VectorSubcoreMesh(
        core_axis_name="c",
        subcore_axis_name="s",
        num_cores=_NUM_CORES,
        num_subcores=_NUM_SUBCORES,
    ),
    scratch_types=[
        pltpu.VMEM((_BPW,), jnp.int32),
        pltpu.VMEM((_ZONES, _DIM), jnp.float32),
        pltpu.VMEM((_DIM, 16), jnp.float32),
        pltpu.VMEM((_DIM, _BPW), jnp.float32),
    ],
    compiler_params=pltpu.CompilerParams(
        needs_layout_passes=False
    ),
)
def _gather_kernel(idx_hbm, table_hbm, out_hbm, idx_v, table_v, ttv, rows_v):
    wid = lax.axis_index("s") * _NUM_CORES + lax.axis_index("c")
    base = wid * _BPW
    pltpu.sync_copy(idx_hbm.at[pl.ds(base, _BPW)], idx_v)
    pltpu.sync_copy(table_hbm, table_v)
    lane = lax.iota(jnp.int32, 16)

    @plsc.parallel_loop(0, _ZONES * (_DIM // 16), 1, unroll=2)
    def setup(t):
        r = t // (_DIM // 16)
        h = t % (_DIM // 16)
        vals = table_v[r, pl.ds(h * 16, 16)]
        plsc.store_scatter(
            ttv, [lane + h * 16, jnp.full((16,), 0, jnp.int32) + r], vals
        )

    @plsc.parallel_loop(0, _CHUNKS, 1, unroll=1)
    def body(i):
        rowidx = idx_v[pl.ds(i * 16, 16)]

        @plsc.parallel_loop(0, _DIM, 1, unroll=4)
        def cols(d):
            tcol = ttv[d]
            rows_v[d, pl.ds(i * 16, 16)] = tcol.at[rowidx].get(
                mode="promise_in_bounds"
            )

    pltpu.sync_copy(rows_v, out_hbm.at[:, pl.ds(base, _BPW)])


def kernel(zone_idx, embedding_table):
    out_t = _gather_kernel(zone_idx.astype(jnp.int32), embedding_table)
    return out_t.T


# speedup vs baseline: 1.2998x; 1.0245x over previous
import functools

import jax
import jax.numpy as jnp
from jax import lax
from jax.experimental import pallas as pl
from jax.experimental.pallas import tpu as pltpu
from jax.experimental.pallas import tpu_sc as plsc

_NUM_CORES = 2
_NUM_SUBCORES = 16
_NW = _NUM_CORES * _NUM_SUBCORES

_BATCH = 16384
_DIM = 32
_ZONES = 9
_BPW = _BATCH // _NW
_CHUNKS = _BPW // 16


@functools.partial(
    pl.kernel,
    out_type=jax.ShapeDtypeStruct((_DIM, _BATCH), jnp.float32),
    mesh=plsc.VectorSubcoreMesh(
        core_axis_name="c",
        subcore_axis_name="s",
        num_cores=_NUM_CORES,
        num_subcores=_NUM_SUBCORES,
    ),
    scratch_types=[
        pltpu.VMEM((_BPW,), jnp.int32),
        pltpu.VMEM((_ZONES, _DIM), jnp.float32),
        pltpu.VMEM((_DIM, 16), jnp.float32),
        pltpu.VMEM((_DIM, _BPW), jnp.float32),
    ],
    compiler_params=pltpu.CompilerParams(
        needs_layout_passes=False
    ),
)
def _gather_kernel(idx_hbm, table_hbm, out_hbm, idx_v, table_v, ttv, rows_v):
    wid = lax.axis_index("s") * _NUM_CORES + lax.axis_index("c")
    base = wid * _BPW
    pltpu.sync_copy(idx_hbm.at[pl.ds(base, _BPW)], idx_v)
    pltpu.sync_copy(table_hbm, table_v)
    lane = lax.iota(jnp.int32, 16)

    @plsc.parallel_loop(0, _ZONES * (_DIM // 16), 1, unroll=2)
    def setup(t):
        r = t // (_DIM // 16)
        h = t % (_DIM // 16)
        vals = table_v[r, pl.ds(h * 16, 16)]
        plsc.store_scatter(
            ttv, [lane + h * 16, jnp.full((16,), 0, jnp.int32) + r], vals
        )

    @plsc.parallel_loop(0, _CHUNKS, 1, unroll=1)
    def body(i):
        rowidx = idx_v[pl.ds(i * 16, 16)]

        @plsc.parallel_loop(0, _DIM, 1, unroll=8)
        def cols(d):
            tcol = ttv[d]
            rows_v[d, pl.ds(i * 16, 16)] = tcol.at[rowidx].get(
                mode="promise_in_bounds"
            )

    pltpu.sync_copy(rows_v, out_hbm.at[:, pl.ds(base, _BPW)])


def kernel(zone_idx, embedding_table):
    out_t = _gather_kernel(zone_idx.astype(jnp.int32), embedding_table)
    return out_t.T


# speedup vs baseline: 1.3106x; 1.0083x over previous
import functools

import jax
import jax.numpy as jnp
from jax import lax
from jax.experimental import pallas as pl
from jax.experimental.pallas import tpu as pltpu
from jax.experimental.pallas import tpu_sc as plsc

_NUM_CORES = 2
_NUM_SUBCORES = 16
_NW = _NUM_CORES * _NUM_SUBCORES

_BATCH = 16384
_DIM = 32
_ZONES = 9
_BPW = _BATCH // _NW
_CHUNKS = _BPW // 16


@functools.partial(
    pl.kernel,
    out_type=jax.ShapeDtypeStruct((_DIM, _BATCH), jnp.float32),
    mesh=plsc.VectorSubcoreMesh(
        core_axis_name="c",
        subcore_axis_name="s",
        num_cores=_NUM_CORES,
        num_subcores=_NUM_SUBCORES,
    ),
    scratch_types=[
        pltpu.VMEM((_BPW,), jnp.int32),
        pltpu.VMEM((_ZONES, _DIM), jnp.float32),
        pltpu.VMEM((_DIM, 16), jnp.float32),
        pltpu.VMEM((_DIM, _BPW), jnp.float32),
    ],
    compiler_params=pltpu.CompilerParams(
        needs_layout_passes=False
    ),
)
def _gather_kernel(idx_hbm, table_hbm, out_hbm, idx_v, table_v, ttv, rows_v):
    wid = lax.axis_index("s") * _NUM_CORES + lax.axis_index("c")
    base = wid * _BPW
    pltpu.sync_copy(idx_hbm.at[pl.ds(base, _BPW)], idx_v)
    pltpu.sync_copy(table_hbm, table_v)
    lane = lax.iota(jnp.int32, 16)

    @plsc.parallel_loop(0, _ZONES * (_DIM // 16), 1, unroll=2)
    def setup(t):
        r = t // (_DIM // 16)
        h = t % (_DIM // 16)
        vals = table_v[r, pl.ds(h * 16, 16)]
        plsc.store_scatter(
            ttv, [lane + h * 16, jnp.full((16,), 0, jnp.int32) + r], vals
        )

    @plsc.parallel_loop(0, _CHUNKS // 2, 1, unroll=1)
    def body(i):
        rowidx0 = idx_v[pl.ds(i * 32, 16)]
        rowidx1 = idx_v[pl.ds(i * 32 + 16, 16)]

        @plsc.parallel_loop(0, _DIM, 1, unroll=8)
        def cols(d):
            tcol = ttv[d]
            rows_v[d, pl.ds(i * 32, 16)] = tcol.at[rowidx0].get(
                mode="promise_in_bounds"
            )
            rows_v[d, pl.ds(i * 32 + 16, 16)] = tcol.at[rowidx1].get(
                mode="promise_in_bounds"
            )

    pltpu.sync_copy(rows_v, out_hbm.at[:, pl.ds(base, _BPW)])


def kernel(zone_idx, embedding_table):
    out_t = _gather_kernel(zone_idx.astype(jnp.int32), embedding_table)
    return out_t.T
